# baseline (speedup 1.0000x reference)
import functools

import jax
import jax.numpy as jnp
from jax import lax
from jax.experimental import pallas as pl
from jax.experimental.pallas import tpu as pltpu
from jax.experimental.pallas import tpu_sc as plsc

L = 16
NC = 2
NS = 16
NW = NC * NS
SP = 80
OFF = 8


def _build_tables(vals_v, a_v, b_v, cd_v, num_knots):
    k = num_knots
    for j in range(SP // L):
        s = lax.iota(jnp.int32, L) + j * L
        i = s - OFF
        ii = jnp.clip(i, 0, k - 2)
        v0 = plsc.load_gather(vals_v, [ii])
        v1 = plsc.load_gather(vals_v, [ii + 1])
        vm = plsc.load_gather(vals_v, [jnp.clip(ii - 1, 0, k - 1)])
        vp = plsc.load_gather(vals_v, [jnp.clip(ii + 2, 0, k - 1)])
        m0h = 0.5 * (v1 - vm)
        m1h = 0.5 * (vp - v0)
        a = v0
        b = m0h
        c = 3.0 * (v1 - v0) - 2.0 * m0h - m1h
        d = 2.0 * (v0 - v1) + m0h + m1h
        fi = i.astype(jnp.float32)
        lin = v1 - v0
        left = i < 0
        right = i >= k - 1
        edge = left | right
        zero = jnp.zeros((L,), jnp.float32)
        a = jnp.where(left, v0 + fi * lin,
                      jnp.where(right, v1 + (fi - (k - 1)) * lin, a))
        b = jnp.where(edge, lin, b)
        c = jnp.where(edge, zero, c)
        d = jnp.where(edge, zero, d)
        a_v[pl.ds(j * L, L)] = a
        b_v[pl.ds(j * L, L)] = b
        cd = plsc.pack(c, d, format=plsc.PackFormat.INTERLEAVED)
        cd_v[pl.ds(j * L, L)] = plsc.bitcast(cd, jnp.int32)


def _make_kernel(n, num_knots, chunk):
    per_w = n // NW
    steps = chunk // L
    chunks = per_w // chunk
    scale = float(num_knots - 1)

    mesh = plsc.VectorSubcoreMesh(core_axis_name="c", subcore_axis_name="s")

    @functools.partial(
        pl.kernel,
        mesh=mesh,
        out_type=jax.ShapeDtypeStruct((n,), jnp.float32),
        compiler_params=pltpu.CompilerParams(needs_layout_passes=False),
        scratch_types=[
            pltpu.VMEM((num_knots,), jnp.float32),
            pltpu.VMEM((SP,), jnp.float32),
            pltpu.VMEM((SP,), jnp.float32),
            pltpu.VMEM((SP,), jnp.int32),
            pltpu.VMEM((chunk,), jnp.float32),
            pltpu.VMEM((chunk,), jnp.float32),
            pltpu.VMEM((chunk,), jnp.float32),
            pltpu.VMEM((chunk,), jnp.float32),
            pltpu.SemaphoreType.DMA,
            pltpu.SemaphoreType.DMA,
            pltpu.SemaphoreType.DMA,
            pltpu.SemaphoreType.DMA,
        ],
    )
    def k(x_hbm, vals_hbm, out_hbm, vals_v, a_v, b_v, cd_v,
          xb0, xb1, yb0, yb1, is0, is1, os0, os1):
        wid = lax.axis_index("s") * NC + lax.axis_index("c")
        pltpu.sync_copy(vals_hbm, vals_v)
        _build_tables(vals_v, a_v, b_v, cd_v, num_knots)
        base = wid * per_w
        bufs = ((xb0, yb0, is0, os0), (xb1, yb1, is1, os1))

        def x_slice(g):
            return x_hbm.at[pl.ds(base + g * chunk, chunk)]

        def y_slice(g):
            return out_hbm.at[pl.ds(base + g * chunk, chunk)]

        def compute(xb, yb):
            @plsc.parallel_loop(0, steps, unroll=4)
            def step(t):
                xv = xb[pl.ds(t * L, L)]
                xs = xv * scale + float(OFF)
                s = xs.astype(jnp.int32)
                u = xs - s.astype(jnp.float32)
                a = plsc.load_gather(a_v, [s])
                b = plsc.load_gather(b_v, [s])
                w = plsc.load_gather(cd_v, [s])
                c, d = plsc.unpack(plsc.bitcast(w, jnp.bfloat16),
                                   format=plsc.PackFormat.INTERLEAVED)
                yb[pl.ds(t * L, L)] = a + u * (b + u * (c + u * d))

        pltpu.async_copy(x_slice(0), xb0, is0)

        def outer(gg, carry):
            for p in range(2):
                xb, yb, isem, osem = bufs[p]
                nxb, _, nisem, _ = bufs[1 - p]
                g = 2 * gg + p

                @pl.when(g + 1 < chunks)
                def _():
                    pltpu.async_copy(x_slice(g + 1), nxb, nisem)

                pltpu.make_async_copy(x_slice(g), xb, isem).wait()

                @pl.when(g >= 2)
                def _():
                    pltpu.make_async_copy(yb, y_slice(g - 2), osem).wait()

                compute(xb, yb)
                pltpu.async_copy(yb, y_slice(g), osem)
            return carry

        lax.fori_loop(0, chunks // 2, outer, 0)
        pltpu.make_async_copy(yb0, y_slice(chunks - 2), os0).wait()
        pltpu.make_async_copy(yb1, y_slice(chunks - 1), os1).wait()

    return k


def kernel(x, values):
    n = x.shape[0]
    num_knots = values.shape[0]
    chunk = 4096
    while n % (NW * chunk * 2) != 0:
        chunk //= 2
    return _make_kernel(n, num_knots, chunk)(x, values)


# speedup vs baseline: 1.0681x; 1.0681x over previous
import functools

import jax
import jax.numpy as jnp
from jax import lax
from jax.experimental import pallas as pl
from jax.experimental.pallas import tpu as pltpu
from jax.experimental.pallas import tpu_sc as plsc

L = 16
NC = 2
NS = 16
NW = NC * NS
SP = 80
OFF = 8


def _build_tables(vals_v, a_v, b_v, cd_v, num_knots):
    k = num_knots
    for j in range(SP // L):
        s = lax.iota(jnp.int32, L) + j * L
        i = s - OFF
        ii = jnp.clip(i, 0, k - 2)
        v0 = plsc.load_gather(vals_v, [ii])
        v1 = plsc.load_gather(vals_v, [ii + 1])
        vm = plsc.load_gather(vals_v, [jnp.clip(ii - 1, 0, k - 1)])
        vp = plsc.load_gather(vals_v, [jnp.clip(ii + 2, 0, k - 1)])
        m0h = 0.5 * (v1 - vm)
        m1h = 0.5 * (vp - v0)
        a = v0
        b = m0h
        c = 3.0 * (v1 - v0) - 2.0 * m0h - m1h
        d = 2.0 * (v0 - v1) + m0h + m1h
        fi = i.astype(jnp.float32)
        lin = v1 - v0
        left = i < 0
        right = i >= k - 1
        edge = left | right
        zero = jnp.zeros((L,), jnp.float32)
        a = jnp.where(left, v0 + fi * lin,
                      jnp.where(right, v1 + (fi - (k - 1)) * lin, a))
        b = jnp.where(edge, lin, b)
        c = jnp.where(edge, zero, c)
        d = jnp.where(edge, zero, d)
        a_v[pl.ds(j * L, L)] = a
        b_v[pl.ds(j * L, L)] = b
        cd = plsc.pack(c, d, format=plsc.PackFormat.INTERLEAVED)
        cd_v[pl.ds(j * L, L)] = plsc.bitcast(cd, jnp.int32)


def _make_kernel(n, num_knots, chunk):
    per_w = n // NW
    steps = chunk // L
    chunks = per_w // chunk
    scale = float(num_knots - 1)

    mesh = plsc.VectorSubcoreMesh(core_axis_name="c", subcore_axis_name="s")

    @functools.partial(
        pl.kernel,
        mesh=mesh,
        out_type=jax.ShapeDtypeStruct((n,), jnp.float32),
        compiler_params=pltpu.CompilerParams(needs_layout_passes=False),
        scratch_types=[
            pltpu.VMEM((num_knots,), jnp.float32),
            pltpu.VMEM((SP,), jnp.float32),
            pltpu.VMEM((SP,), jnp.float32),
            pltpu.VMEM((SP,), jnp.int32),
            pltpu.VMEM((chunk,), jnp.float32),
            pltpu.VMEM((chunk,), jnp.float32),
            pltpu.VMEM((chunk,), jnp.float32),
            pltpu.VMEM((chunk,), jnp.float32),
            pltpu.SemaphoreType.DMA,
            pltpu.SemaphoreType.DMA,
            pltpu.SemaphoreType.DMA,
            pltpu.SemaphoreType.DMA,
        ],
    )
    def k(x_hbm, vals_hbm, out_hbm, vals_v, a_v, b_v, cd_v,
          xb0, xb1, yb0, yb1, is0, is1, os0, os1):
        wid = lax.axis_index("s") * NC + lax.axis_index("c")
        pltpu.sync_copy(vals_hbm, vals_v)
        _build_tables(vals_v, a_v, b_v, cd_v, num_knots)
        base = wid * per_w
        bufs = ((xb0, yb0, is0, os0), (xb1, yb1, is1, os1))

        def x_slice(g):
            return x_hbm.at[pl.ds(base + g * chunk, chunk)]

        def y_slice(g):
            return out_hbm.at[pl.ds(base + g * chunk, chunk)]

        def compute(xb, yb):
            @plsc.parallel_loop(0, steps, unroll=4)
            def step(t):
                xv = xb[pl.ds(t * L, L)]
                xs = xv * scale + float(OFF)
                s = xs.astype(jnp.int32)
                u = xs - s.astype(jnp.float32)
                a = plsc.load_gather(a_v, [s])
                b = plsc.load_gather(b_v, [s])
                w = plsc.load_gather(cd_v, [s])
                c, d = plsc.unpack(plsc.bitcast(w, jnp.bfloat16),
                                   format=plsc.PackFormat.INTERLEAVED)
                yb[pl.ds(t * L, L)] = a + u * (b + u * (c + u * d))

        pltpu.async_copy(x_slice(0), xb0, is0)

        def outer(gg, carry):
            for p in range(2):
                xb, yb, isem, osem = bufs[p]
                nxb, _, nisem, _ = bufs[1 - p]
                g = 2 * gg + p

                @pl.when(g + 1 < chunks)
                def _():
                    pltpu.async_copy(x_slice(g + 1), nxb, nisem)

                pltpu.make_async_copy(x_slice(g), xb, isem).wait()

                @pl.when(g >= 2)
                def _():
                    pltpu.make_async_copy(yb, y_slice(g - 2), osem).wait()

                compute(xb, yb)
                pltpu.async_copy(yb, y_slice(g), osem)
            return carry

        lax.fori_loop(0, chunks // 2, outer, 0)
        pltpu.make_async_copy(yb0, y_slice(chunks - 2), os0).wait()
        pltpu.make_async_copy(yb1, y_slice(chunks - 1), os1).wait()

    return k


def kernel(x, values):
    n = x.shape[0]
    num_knots = values.shape[0]
    chunk = 8192
    while n % (NW * chunk * 2) != 0:
        chunk //= 2
    return _make_kernel(n, num_knots, chunk)(x, values)
